# baseline (speedup 1.0000x reference)
import dataclasses
import functools

import jax
import jax.numpy as jnp
from jax import lax
from jax.experimental import pallas as pl
from jax.experimental.pallas import tpu as pltpu
from jax.experimental.pallas import tpu_sc as plsc

N = 10000
D = 128
E = 320000
SCALE = float(D) ** -0.5

NC = 2
NS = 16
L = 16
NW = NC * NS
B = 32
KSB = 12
NSB = 28
QB = 512
NPAD = 10240
BLK = KSB * B
E_PAD = NW * NSB * BLK
NDEN = 10240

ROW_BLK = 1000


def _qkv_body(x_ref, wq_ref, bq_ref, wk_ref, bk_ref, wv_ref, bv_ref,
              q_ref, kv_ref):
    xb = x_ref[...]
    mm = functools.partial(jnp.dot, preferred_element_type=jnp.float32,
                           precision=lax.Precision.HIGHEST)
    q_ref[...] = mm(xb, wq_ref[...].T) + bq_ref[...]
    k = mm(xb, wk_ref[...].T) + bk_ref[...]
    v = mm(xb, wv_ref[...].T) + bv_ref[...]
    kb = jax.lax.bitcast_convert_type(k, jnp.uint32) + jnp.uint32(0x8000)
    vb = jax.lax.bitcast_convert_type(v, jnp.uint32) + jnp.uint32(0x8000)
    kv_ref[...] = jax.lax.bitcast_convert_type(
        (vb & jnp.uint32(0xFFFF0000)) | (kb >> 16), jnp.int32)


def _qkv(x_pad, Wq, bq, Wk, bk, Wv, bv):
    full = lambda shape: pl.BlockSpec(shape, lambda i: (0,) * len(shape))
    return pl.pallas_call(
        _qkv_body,
        grid=(NPAD // QB,),
        in_specs=[pl.BlockSpec((QB, D), lambda i: (i, 0)),
                  full((D, D)), full((1, D)), full((D, D)),
                  full((1, D)), full((D, D)), full((1, D))],
        out_specs=[pl.BlockSpec((QB, D), lambda i: (i, 0)),
                   pl.BlockSpec((QB, D), lambda i: (i, 0))],
        out_shape=[jax.ShapeDtypeStruct((NPAD, D), jnp.float32),
                   jax.ShapeDtypeStruct((NPAD, D), jnp.int32)],
    )(x_pad, Wq, bq.reshape(1, D), Wk, bk.reshape(1, D), Wv,
      bv.reshape(1, D))


def _pe_body(ew_ref, c_ref, pe_ref):
    pe_ref[...] = jnp.exp(ew_ref[...] * c_ref[0] + c_ref[1])


def _pe(edge_weight, We, be):
    ew2d = edge_weight.reshape(E // D, D)
    coefs = jnp.concatenate([We.reshape(1), be.reshape(1)])
    out = pl.pallas_call(
        _pe_body,
        in_specs=[
            pl.BlockSpec(memory_space=pltpu.VMEM),
            pl.BlockSpec(memory_space=pltpu.SMEM),
        ],
        out_shape=jax.ShapeDtypeStruct((E // D, D), jnp.float32),
    )(ew2d, coefs)
    return out.reshape(E)


def _edge_body(q_hbm, kv_hbm, src_hbm, dst_hbm, pe_hbm,
               msg_hbm, den0_hbm, den1_hbm,
               sblk0, sblk1, dblk0, dblk1, pblk0, pblk1,
               qb0, qb1, qb2, kvb0, kvb1, kvb2, m0, m1, m2,
               es0, es1, es2, ix0, ix1, ix2,
               acc_msg, acc_den,
               gsem0, gsem1, gsem2, ssem0, ssem1, ssem2, bsem0, bsem1):
    c = lax.axis_index("c")
    s = lax.axis_index("s")
    wid = c * NS + s
    tile_base = wid * (NSB * BLK)

    sblk = (sblk0, sblk1)
    dblk = (dblk0, dblk1)
    pblk = (pblk0, pblk1)
    qbuf = (qb0, qb1, qb2)
    kvbuf = (kvb0, kvb1, kvb2)
    mr = (m0, m1, m2)
    es = (es0, es1, es2)
    ix = (ix0, ix1, ix2)
    gsem = (gsem0, gsem1, gsem2)
    ssem = (ssem0, ssem1, ssem2)
    bsem = (bsem0, bsem1)

    lane = jax.lax.iota(jnp.int32, L)
    zero16 = jnp.zeros((L,), jnp.float32)

    @pl.loop(0, B)
    def _(r):
        for cc in range(D // L):
            m0[r, pl.ds(cc * L, L)] = zero16
    for cc in range(B // L):
        es0[pl.ds(cc * L, L)] = zero16

    for j in range(640 // B):
        pltpu.async_copy(m0, acc_msg.at[pl.ds(s * 624 + j * B, B)], gsem0)
    for j in range(640 // B):
        pltpu.make_async_copy(q_hbm.at[pl.ds(0, B)], m0, gsem0).wait()

    @pl.when(s < 10)
    def _():
        for j in range(1024 // B):
            pltpu.async_copy(es0, acc_den.at[pl.ds(s * 1024 + j * B, B)],
                             gsem1)
        for j in range(1024 // B):
            pltpu.make_async_copy(pe_hbm.at[pl.ds(0, B)], es0, gsem1).wait()

    plsc.subcore_barrier()

    def issue_block(sb, t):
        off = tile_base + sb * BLK
        pltpu.async_copy(src_hbm.at[pl.ds(off, BLK)], sblk[t], bsem[t])
        pltpu.async_copy(dst_hbm.at[pl.ds(off, BLK)], dblk[t], bsem[t])
        pltpu.async_copy(pe_hbm.at[pl.ds(off, BLK)], pblk[t], bsem[t])

    def wait_block(t):
        pltpu.make_async_copy(src_hbm.at[pl.ds(0, BLK)], sblk[t],
                              bsem[t]).wait()
        pltpu.make_async_copy(src_hbm.at[pl.ds(0, BLK)], dblk[t],
                              bsem[t]).wait()
        pltpu.make_async_copy(pe_hbm.at[pl.ds(0, BLK)], pblk[t],
                              bsem[t]).wait()

    def issue_gathers(kb, t, p):
        pltpu.async_copy(q_hbm.at[dblk[t].at[pl.ds(kb * B, B)]], qbuf[p],
                         gsem[p])
        pltpu.async_copy(kv_hbm.at[sblk[t].at[pl.ds(kb * B, B)]], kvbuf[p],
                         gsem[p])

    def wait_gathers(p):
        pltpu.make_async_copy(q_hbm.at[pl.ds(0, B)], qbuf[p],
                              gsem[p]).wait()
        pltpu.make_async_copy(kv_hbm.at[pl.ds(0, B)], kvbuf[p],
                              gsem[p]).wait()

    def issue_scatters(p):
        pltpu.async_copy(mr[p], acc_msg.at[ix[p]], ssem[p], add=True)
        pltpu.async_copy(es[p], acc_den.at[ix[p]], ssem[p], add=True)

    def wait_scatters(p):
        pltpu.make_async_copy(q_hbm.at[pl.ds(0, B)], mr[p], ssem[p]).wait()
        pltpu.make_async_copy(pe_hbm.at[pl.ds(0, B)], es[p], ssem[p]).wait()

    def compute(kb, t, p):
        qb = qbuf[p]
        kvb = kvbuf[p]

        @pl.loop(0, B // L)
        def _(g):
            svec = jnp.zeros((L,), jnp.float32)
            for e in range(L):
                row = g * L + e
                a16 = jnp.zeros((L,), jnp.float32)
                for ch in range(D // L):
                    w = kvb[row, pl.ds(ch * L, L)]
                    kf = plsc.bitcast(w << 16, jnp.float32)
                    a16 = a16 + qb[row, pl.ds(ch * L, L)] * kf
                svec = jnp.where(lane == e, jnp.sum(a16), svec)
            esv = (jnp.exp(svec * SCALE) *
                   pblk[t][pl.ds(kb * B + g * L, L)])
            es[p][pl.ds(g * L, L)] = esv
            ix[p][pl.ds(g * L, L)] = dblk[t][pl.ds(kb * B + g * L, L)]
            for e in range(L):
                row = g * L + e
                se = jnp.broadcast_to(
                    jnp.sum(jnp.where(lane == e, esv, 0.0)), (L,))
                for ch in range(D // L):
                    w = kvb[row, pl.ds(ch * L, L)]
                    vf = plsc.bitcast(w & jnp.int32(-65536), jnp.float32)
                    mr[p][row, pl.ds(ch * L, L)] = vf * se

    issue_block(0, 0)
    issue_block(1, 1)
    wait_block(0)
    for pp in range(3):
        issue_gathers(pp, 0, pp)

    @pl.loop(0, NSB // 2)
    def _(sbj):
        for t in (0, 1):
            sbi = sbj * 2 + t

            @pl.loop(0, KSB // 3)
            def _(jj):
                for p in (0, 1, 2):
                    i = jj * 3 + p
                    gi = sbi * KSB + i
                    wait_gathers(p)

                    @pl.when(gi >= 3)
                    def _():
                        wait_scatters(p)

                    compute(i, t, p)
                    issue_scatters(p)

                    @pl.when(jj < KSB // 3 - 1)
                    def _():
                        issue_gathers(i + 3, t, p)

            @pl.when(sbi + 1 < NSB)
            def _():
                wait_block(1 - t)
                for pp in range(3):
                    issue_gathers(pp, 1 - t, pp)

            @pl.when(sbi + 2 < NSB)
            def _():
                issue_block(sbi + 2, t)

    for pp in range(3):
        wait_scatters(pp)

    plsc.subcore_barrier()

    pltpu.sync_copy(acc_msg.at[pl.ds(s * 624, 624)],
                    msg_hbm.at[c, pl.ds(s * 624, 624)])

    @pl.when(s == NS - 1)
    def _():
        pltpu.sync_copy(acc_msg.at[pl.ds(9984, 16)],
                        msg_hbm.at[c, pl.ds(9984, 16)])

    @pl.when(s < 10)
    def _():
        @pl.when(c == 0)
        def _():
            pltpu.sync_copy(acc_den.at[pl.ds(s * 1024, 1024)],
                            den0_hbm.at[pl.ds(s * 1024, 1024)])

        @pl.when(c == 1)
        def _():
            pltpu.sync_copy(acc_den.at[pl.ds(s * 1024, 1024)],
                            den1_hbm.at[pl.ds(s * 1024, 1024)])


def _sc_compiler_params():
    cp = pltpu.CompilerParams()
    if "needs_layout_passes" in pltpu.CompilerParams.__dataclass_fields__:
        cp = dataclasses.replace(cp, needs_layout_passes=False)
    return cp


def _edge(q, kv, src, dst, pe):
    mesh = plsc.VectorSubcoreMesh(core_axis_name="c", subcore_axis_name="s")
    kern = pl.kernel(
        _edge_body,
        out_type=[
            jax.ShapeDtypeStruct((NC, N, D), jnp.float32),
            jax.ShapeDtypeStruct((NDEN,), jnp.float32),
            jax.ShapeDtypeStruct((NDEN,), jnp.float32),
        ],
        mesh=mesh,
        compiler_params=_sc_compiler_params(),
        scratch_types=(
            [pltpu.VMEM((BLK,), jnp.int32)] * 4 +
            [pltpu.VMEM((BLK,), jnp.float32)] * 2 +
            [pltpu.VMEM((B, D), jnp.float32)] * 3 +
            [pltpu.VMEM((B, D), jnp.int32)] * 3 +
            [pltpu.VMEM((B, D), jnp.float32)] * 3 +
            [pltpu.VMEM((B,), jnp.float32)] * 3 +
            [pltpu.VMEM((B,), jnp.int32)] * 3 +
            [pltpu.VMEM_SHARED((N, D), jnp.float32),
             pltpu.VMEM_SHARED((NDEN,), jnp.float32)] +
            [pltpu.SemaphoreType.DMA] * 8
        ),
    )
    return kern(q, kv, src, dst, pe)


def _ln(h, g_ref, b_ref):
    mu = jnp.mean(h, axis=-1, keepdims=True)
    var = jnp.mean((h - mu) ** 2, axis=-1, keepdims=True)
    return (h - mu) * jax.lax.rsqrt(var + 1e-5) * g_ref[...] + b_ref[...]


def _post_body(x_ref, p_ref, d0_ref, d1_ref, wself_ref, bself_ref,
               g1_ref, b1_ref, g2_ref, b2_ref, wf1_ref, bf1_ref,
               wf2_ref, bf2_ref, y_ref):
    xb = x_ref[...]
    mm = functools.partial(jnp.dot, preferred_element_type=jnp.float32,
                           precision=lax.Precision.HIGHEST)
    num = p_ref[0] + p_ref[1]
    den = d0_ref[...] + d1_ref[...]
    out = num / jnp.clip(den, 1e-12, None)
    h = _ln(xb + out + mm(xb, wself_ref[...].T) + bself_ref[...],
            g1_ref, b1_ref)
    f1 = mm(h, wf1_ref[...].T) + bf1_ref[...]
    gelu = 0.5 * f1 * (1.0 + lax.erf(f1 * (2.0 ** -0.5)))
    f2 = mm(gelu, wf2_ref[...].T) + bf2_ref[...]
    y_ref[...] = _ln(h + f2, g2_ref, b2_ref)


def _post(x, msg, den0, den1, Wself, bself, ln1_g, ln1_b, ln2_g, ln2_b,
          Wf1, bf1, Wf2, bf2):
    full = lambda shape: pl.BlockSpec(shape, lambda i: (0,) * len(shape))
    return pl.pallas_call(
        _post_body,
        grid=(N // ROW_BLK,),
        in_specs=[
            pl.BlockSpec((ROW_BLK, D), lambda i: (i, 0)),
            pl.BlockSpec((NC, ROW_BLK, D), lambda i: (0, i, 0)),
            pl.BlockSpec((ROW_BLK, 1), lambda i: (i, 0)),
            pl.BlockSpec((ROW_BLK, 1), lambda i: (i, 0)),
            full((D, D)), full((1, D)),
            full((1, D)), full((1, D)),
            full((1, D)), full((1, D)),
            full((2 * D, D)), full((1, 2 * D)),
            full((D, 2 * D)), full((1, D)),
        ],
        out_specs=pl.BlockSpec((ROW_BLK, D), lambda i: (i, 0)),
        out_shape=jax.ShapeDtypeStruct((N, D), jnp.float32),
    )(x, msg, den0, den1, Wself, bself.reshape(1, D),
      ln1_g.reshape(1, D), ln1_b.reshape(1, D),
      ln2_g.reshape(1, D), ln2_b.reshape(1, D),
      Wf1, bf1.reshape(1, 2 * D), Wf2, bf2.reshape(1, D))


def kernel(x, edge_index, edge_weight, Wq, bq, Wk, bk, Wv, bv, Wself, bself,
           We, be, ln1_g, ln1_b, ln2_g, ln2_b, Wf1, bf1, Wf2, bf2):
    x_pad = jnp.pad(x, ((0, NPAD - N), (0, 0)))
    q, kv = _qkv(x_pad, Wq, bq, Wk, bk, Wv, bv)
    pe = _pe(edge_weight, We, be)

    pad = E_PAD - E
    src = jnp.pad(edge_index[0], (0, pad))
    dst = jnp.pad(edge_index[1], (0, pad))
    pe_pad = jnp.pad(pe, (0, pad))

    msg, den0, den1 = _edge(q, kv, src, dst, pe_pad)
    den0c = den0[:N].reshape(N, 1)
    den1c = den1[:N].reshape(N, 1)

    return _post(x, msg, den0c, den1c, Wself, bself, ln1_g, ln1_b,
                 ln2_g, ln2_b, Wf1, bf1, Wf2, bf2)


# speedup vs baseline: 1.0056x; 1.0056x over previous
import dataclasses
import functools

import jax
import jax.numpy as jnp
from jax import lax
from jax.experimental import pallas as pl
from jax.experimental.pallas import tpu as pltpu
from jax.experimental.pallas import tpu_sc as plsc

N = 10000
D = 128
E = 320000
SCALE = float(D) ** -0.5

NC = 2
NS = 16
L = 16
NW = NC * NS
B = 48
KSB = 8
NSB = 28
QB = 512
NPAD = 10240
BLK = KSB * B
E_PAD = NW * NSB * BLK
NDEN = 10240

ROW_BLK = 1000


def _qkv_body(x_ref, wq_ref, bq_ref, wk_ref, bk_ref, wv_ref, bv_ref,
              q_ref, kv_ref):
    xb = x_ref[...]
    mm = functools.partial(jnp.dot, preferred_element_type=jnp.float32,
                           precision=lax.Precision.HIGHEST)
    q_ref[...] = mm(xb, wq_ref[...].T) + bq_ref[...]
    k = mm(xb, wk_ref[...].T) + bk_ref[...]
    v = mm(xb, wv_ref[...].T) + bv_ref[...]
    kb = jax.lax.bitcast_convert_type(k, jnp.uint32) + jnp.uint32(0x8000)
    vb = jax.lax.bitcast_convert_type(v, jnp.uint32) + jnp.uint32(0x8000)
    kv_ref[...] = jax.lax.bitcast_convert_type(
        (vb & jnp.uint32(0xFFFF0000)) | (kb >> 16), jnp.int32)


def _qkv(x_pad, Wq, bq, Wk, bk, Wv, bv):
    full = lambda shape: pl.BlockSpec(shape, lambda i: (0,) * len(shape))
    return pl.pallas_call(
        _qkv_body,
        grid=(NPAD // QB,),
        in_specs=[pl.BlockSpec((QB, D), lambda i: (i, 0)),
                  full((D, D)), full((1, D)), full((D, D)),
                  full((1, D)), full((D, D)), full((1, D))],
        out_specs=[pl.BlockSpec((QB, D), lambda i: (i, 0)),
                   pl.BlockSpec((QB, D), lambda i: (i, 0))],
        out_shape=[jax.ShapeDtypeStruct((NPAD, D), jnp.float32),
                   jax.ShapeDtypeStruct((NPAD, D), jnp.int32)],
    )(x_pad, Wq, bq.reshape(1, D), Wk, bk.reshape(1, D), Wv,
      bv.reshape(1, D))


def _pe_body(ew_ref, c_ref, pe_ref):
    pe_ref[...] = jnp.exp(ew_ref[...] * c_ref[0] + c_ref[1])


def _pe(edge_weight, We, be):
    ew2d = edge_weight.reshape(E // D, D)
    coefs = jnp.concatenate([We.reshape(1), be.reshape(1)])
    out = pl.pallas_call(
        _pe_body,
        in_specs=[
            pl.BlockSpec(memory_space=pltpu.VMEM),
            pl.BlockSpec(memory_space=pltpu.SMEM),
        ],
        out_shape=jax.ShapeDtypeStruct((E // D, D), jnp.float32),
    )(ew2d, coefs)
    return out.reshape(E)


def _edge_body(q_hbm, kv_hbm, src_hbm, dst_hbm, pe_hbm,
               msg_hbm, den0_hbm, den1_hbm,
               sblk0, sblk1, dblk0, dblk1, pblk0, pblk1,
               qb0, qb1, kvb0, kvb1, m0, m1, es0, es1, ix0, ix1,
               acc_msg, acc_den,
               gsem0, gsem1, ssem0, ssem1, bsem0, bsem1):
    c = lax.axis_index("c")
    s = lax.axis_index("s")
    wid = c * NS + s
    tile_base = wid * (NSB * BLK)

    sblk = (sblk0, sblk1)
    dblk = (dblk0, dblk1)
    pblk = (pblk0, pblk1)
    qbuf = (qb0, qb1)
    kvbuf = (kvb0, kvb1)
    mr = (m0, m1)
    es = (es0, es1)
    ix = (ix0, ix1)
    gsem = (gsem0, gsem1)
    ssem = (ssem0, ssem1)
    bsem = (bsem0, bsem1)

    lane = jax.lax.iota(jnp.int32, L)
    zero16 = jnp.zeros((L,), jnp.float32)

    @pl.loop(0, B)
    def _(r):
        for cc in range(D // L):
            m0[r, pl.ds(cc * L, L)] = zero16
    for cc in range(B // L):
        es0[pl.ds(cc * L, L)] = zero16

    for j in range(40):
        pltpu.async_copy(m0.at[pl.ds(0, L)],
                         acc_msg.at[pl.ds(s * 624 + j * L, L)], gsem0)
    for j in range(40):
        pltpu.make_async_copy(q_hbm.at[pl.ds(0, L)], m0.at[pl.ds(0, L)],
                              gsem0).wait()
    for j in range(40):
        pltpu.async_copy(es0.at[pl.ds(0, L)],
                         acc_den.at[pl.ds(s * 624 + j * L, L)], gsem1)
    for j in range(40):
        pltpu.make_async_copy(pe_hbm.at[pl.ds(0, L)], es0.at[pl.ds(0, L)],
                              gsem1).wait()

    plsc.subcore_barrier()

    def issue_block(sb, t):
        off = tile_base + sb * BLK
        pltpu.async_copy(src_hbm.at[pl.ds(off, BLK)], sblk[t], bsem[t])
        pltpu.async_copy(dst_hbm.at[pl.ds(off, BLK)], dblk[t], bsem[t])
        pltpu.async_copy(pe_hbm.at[pl.ds(off, BLK)], pblk[t], bsem[t])

    def wait_block(t):
        pltpu.make_async_copy(src_hbm.at[pl.ds(0, BLK)], sblk[t],
                              bsem[t]).wait()
        pltpu.make_async_copy(src_hbm.at[pl.ds(0, BLK)], dblk[t],
                              bsem[t]).wait()
        pltpu.make_async_copy(pe_hbm.at[pl.ds(0, BLK)], pblk[t],
                              bsem[t]).wait()

    def issue_gathers(kb, t, p):
        pltpu.async_copy(q_hbm.at[dblk[t].at[pl.ds(kb * B, B)]], qbuf[p],
                         gsem[p])
        pltpu.async_copy(kv_hbm.at[sblk[t].at[pl.ds(kb * B, B)]], kvbuf[p],
                         gsem[p])

    def wait_gathers(p):
        pltpu.make_async_copy(q_hbm.at[pl.ds(0, B)], qbuf[p],
                              gsem[p]).wait()
        pltpu.make_async_copy(kv_hbm.at[pl.ds(0, B)], kvbuf[p],
                              gsem[p]).wait()

    def issue_scatters(p):
        pltpu.async_copy(mr[p], acc_msg.at[ix[p]], ssem[p], add=True)
        pltpu.async_copy(es[p], acc_den.at[ix[p]], ssem[p], add=True)

    def wait_scatters(p):
        pltpu.make_async_copy(q_hbm.at[pl.ds(0, B)], mr[p], ssem[p]).wait()
        pltpu.make_async_copy(pe_hbm.at[pl.ds(0, B)], es[p], ssem[p]).wait()

    def compute(kb, t, p):
        qb = qbuf[p]
        kvb = kvbuf[p]

        @pl.loop(0, B // L)
        def _(g):
            svec = jnp.zeros((L,), jnp.float32)
            for e in range(L):
                row = g * L + e
                a16 = jnp.zeros((L,), jnp.float32)
                for ch in range(D // L):
                    w = kvb[row, pl.ds(ch * L, L)]
                    kf = plsc.bitcast(w << 16, jnp.float32)
                    a16 = a16 + qb[row, pl.ds(ch * L, L)] * kf
                svec = jnp.where(lane == e, jnp.sum(a16), svec)
            esv = (jnp.exp(svec * SCALE) *
                   pblk[t][pl.ds(kb * B + g * L, L)])
            es[p][pl.ds(g * L, L)] = esv
            ix[p][pl.ds(g * L, L)] = dblk[t][pl.ds(kb * B + g * L, L)]
            for e in range(L):
                row = g * L + e
                se = jnp.broadcast_to(
                    jnp.sum(jnp.where(lane == e, esv, 0.0)), (L,))
                for ch in range(D // L):
                    w = kvb[row, pl.ds(ch * L, L)]
                    vf = plsc.bitcast(w & jnp.int32(-65536), jnp.float32)
                    mr[p][row, pl.ds(ch * L, L)] = vf * se

    issue_block(0, 0)
    issue_block(1, 1)
    wait_block(0)
    issue_gathers(0, 0, 0)
    issue_gathers(1, 0, 1)

    @pl.loop(0, NSB // 2)
    def _(sbj):
        for t in (0, 1):
            sbi = sbj * 2 + t

            @pl.loop(0, KSB // 2)
            def _(jj):
                for p in (0, 1):
                    i = jj * 2 + p
                    gi = sbi * KSB + i
                    wait_gathers(p)

                    @pl.when(gi >= 2)
                    def _():
                        wait_scatters(p)

                    compute(i, t, p)
                    issue_scatters(p)

                    @pl.when(jj < KSB // 2 - 1)
                    def _():
                        issue_gathers(i + 2, t, p)

            @pl.when(sbi + 1 < NSB)
            def _():
                wait_block(1 - t)
                issue_gathers(0, 1 - t, 0)
                issue_gathers(1, 1 - t, 1)

            @pl.when(sbi + 2 < NSB)
            def _():
                issue_block(sbi + 2, t)

    wait_scatters(0)
    wait_scatters(1)

    plsc.subcore_barrier()

    pltpu.sync_copy(acc_msg.at[pl.ds(s * 624, 624)],
                    msg_hbm.at[c, pl.ds(s * 624, 624)])

    @pl.when(s == NS - 1)
    def _():
        pltpu.sync_copy(acc_msg.at[pl.ds(9984, 16)],
                        msg_hbm.at[c, pl.ds(9984, 16)])

    @pl.when(s < 10)
    def _():
        @pl.when(c == 0)
        def _():
            pltpu.sync_copy(acc_den.at[pl.ds(s * 1024, 1024)],
                            den0_hbm.at[pl.ds(s * 1024, 1024)])

        @pl.when(c == 1)
        def _():
            pltpu.sync_copy(acc_den.at[pl.ds(s * 1024, 1024)],
                            den1_hbm.at[pl.ds(s * 1024, 1024)])


def _sc_compiler_params():
    cp = pltpu.CompilerParams()
    if "needs_layout_passes" in pltpu.CompilerParams.__dataclass_fields__:
        cp = dataclasses.replace(cp, needs_layout_passes=False)
    return cp


def _edge(q, kv, src, dst, pe):
    mesh = plsc.VectorSubcoreMesh(core_axis_name="c", subcore_axis_name="s")
    kern = pl.kernel(
        _edge_body,
        out_type=[
            jax.ShapeDtypeStruct((NC, N, D), jnp.float32),
            jax.ShapeDtypeStruct((NDEN,), jnp.float32),
            jax.ShapeDtypeStruct((NDEN,), jnp.float32),
        ],
        mesh=mesh,
        compiler_params=_sc_compiler_params(),
        scratch_types=(
            [pltpu.VMEM((BLK,), jnp.int32)] * 4 +
            [pltpu.VMEM((BLK,), jnp.float32)] * 2 +
            [pltpu.VMEM((B, D), jnp.float32)] * 2 +
            [pltpu.VMEM((B, D), jnp.int32)] * 2 +
            [pltpu.VMEM((B, D), jnp.float32)] * 2 +
            [pltpu.VMEM((B,), jnp.float32)] * 2 +
            [pltpu.VMEM((B,), jnp.int32)] * 2 +
            [pltpu.VMEM_SHARED((N, D), jnp.float32),
             pltpu.VMEM_SHARED((NDEN,), jnp.float32)] +
            [pltpu.SemaphoreType.DMA] * 6
        ),
    )
    return kern(q, kv, src, dst, pe)


def _ln(h, g_ref, b_ref):
    mu = jnp.mean(h, axis=-1, keepdims=True)
    var = jnp.mean((h - mu) ** 2, axis=-1, keepdims=True)
    return (h - mu) * jax.lax.rsqrt(var + 1e-5) * g_ref[...] + b_ref[...]


def _post_body(x_ref, p_ref, d0_ref, d1_ref, wself_ref, bself_ref,
               g1_ref, b1_ref, g2_ref, b2_ref, wf1_ref, bf1_ref,
               wf2_ref, bf2_ref, y_ref):
    xb = x_ref[...]
    mm = functools.partial(jnp.dot, preferred_element_type=jnp.float32,
                           precision=lax.Precision.HIGHEST)
    num = p_ref[0] + p_ref[1]
    den = d0_ref[...] + d1_ref[...]
    out = num / jnp.clip(den, 1e-12, None)
    h = _ln(xb + out + mm(xb, wself_ref[...].T) + bself_ref[...],
            g1_ref, b1_ref)
    f1 = mm(h, wf1_ref[...].T) + bf1_ref[...]
    gelu = 0.5 * f1 * (1.0 + lax.erf(f1 * (2.0 ** -0.5)))
    f2 = mm(gelu, wf2_ref[...].T) + bf2_ref[...]
    y_ref[...] = _ln(h + f2, g2_ref, b2_ref)


def _post(x, msg, den0, den1, Wself, bself, ln1_g, ln1_b, ln2_g, ln2_b,
          Wf1, bf1, Wf2, bf2):
    full = lambda shape: pl.BlockSpec(shape, lambda i: (0,) * len(shape))
    return pl.pallas_call(
        _post_body,
        grid=(N // ROW_BLK,),
        in_specs=[
            pl.BlockSpec((ROW_BLK, D), lambda i: (i, 0)),
            pl.BlockSpec((NC, ROW_BLK, D), lambda i: (0, i, 0)),
            pl.BlockSpec((ROW_BLK, 1), lambda i: (i, 0)),
            pl.BlockSpec((ROW_BLK, 1), lambda i: (i, 0)),
            full((D, D)), full((1, D)),
            full((1, D)), full((1, D)),
            full((1, D)), full((1, D)),
            full((2 * D, D)), full((1, 2 * D)),
            full((D, 2 * D)), full((1, D)),
        ],
        out_specs=pl.BlockSpec((ROW_BLK, D), lambda i: (i, 0)),
        out_shape=jax.ShapeDtypeStruct((N, D), jnp.float32),
    )(x, msg, den0, den1, Wself, bself.reshape(1, D),
      ln1_g.reshape(1, D), ln1_b.reshape(1, D),
      ln2_g.reshape(1, D), ln2_b.reshape(1, D),
      Wf1, bf1.reshape(1, 2 * D), Wf2, bf2.reshape(1, D))


def kernel(x, edge_index, edge_weight, Wq, bq, Wk, bk, Wv, bv, Wself, bself,
           We, be, ln1_g, ln1_b, ln2_g, ln2_b, Wf1, bf1, Wf2, bf2):
    x_pad = jnp.pad(x, ((0, NPAD - N), (0, 0)))
    q, kv = _qkv(x_pad, Wq, bq, Wk, bk, Wv, bv)
    pe = _pe(edge_weight, We, be)

    pad = E_PAD - E
    src = jnp.pad(edge_index[0], (0, pad))
    dst = jnp.pad(edge_index[1], (0, pad))
    pe_pad = jnp.pad(pe, (0, pad))

    msg, den0, den1 = _edge(q, kv, src, dst, pe_pad)
    den0c = den0[:N].reshape(N, 1)
    den1c = den1[:N].reshape(N, 1)

    return _post(x, msg, den0c, den1c, Wself, bself, ln1_g, ln1_b,
                 ln2_g, ln2_b, Wf1, bf1, Wf2, bf2)


# speedup vs baseline: 3.5111x; 3.4916x over previous
import dataclasses
import functools

import jax
import jax.numpy as jnp
from jax import lax
from jax.experimental import pallas as pl
from jax.experimental.pallas import tpu as pltpu
from jax.experimental.pallas import tpu_sc as plsc

N = 10000
D = 128
E = 320000
SCALE = float(D) ** -0.5

NC = 2
NS = 16
L = 16
NW = NC * NS
B = 48
KSB = 8
NSB = 28
QB = 512
NPAD = 10240
BLK = KSB * B
E_PAD = NW * NSB * BLK
NDEN = 10240

ROW_BLK = 1000


def _qkv_body(x_ref, wq_ref, bq_ref, wk_ref, bk_ref, wv_ref, bv_ref,
              q_ref, kv_ref):
    xb = x_ref[...]
    mm = functools.partial(jnp.dot, preferred_element_type=jnp.float32,
                           precision=lax.Precision.HIGHEST)
    q_ref[...] = mm(xb, wq_ref[...].T) + bq_ref[...]
    k = mm(xb, wk_ref[...].T) + bk_ref[...]
    v = mm(xb, wv_ref[...].T) + bv_ref[...]
    kb = jax.lax.bitcast_convert_type(k, jnp.uint32) + jnp.uint32(0x8000)
    vb = jax.lax.bitcast_convert_type(v, jnp.uint32) + jnp.uint32(0x8000)
    kv_ref[...] = jax.lax.bitcast_convert_type(
        (vb & jnp.uint32(0xFFFF0000)) | (kb >> 16), jnp.int32)


def _qkv(x_pad, Wq, bq, Wk, bk, Wv, bv):
    full = lambda shape: pl.BlockSpec(shape, lambda i: (0,) * len(shape))
    return pl.pallas_call(
        _qkv_body,
        grid=(NPAD // QB,),
        in_specs=[pl.BlockSpec((QB, D), lambda i: (i, 0)),
                  full((D, D)), full((1, D)), full((D, D)),
                  full((1, D)), full((D, D)), full((1, D))],
        out_specs=[pl.BlockSpec((QB, D), lambda i: (i, 0)),
                   pl.BlockSpec((QB, D), lambda i: (i, 0))],
        out_shape=[jax.ShapeDtypeStruct((NPAD, D), jnp.float32),
                   jax.ShapeDtypeStruct((NPAD, D), jnp.int32)],
    )(x_pad, Wq, bq.reshape(1, D), Wk, bk.reshape(1, D), Wv,
      bv.reshape(1, D))


def _pe_body(ew_ref, c_ref, pe_ref):
    pe_ref[...] = jnp.exp(ew_ref[...] * c_ref[0] + c_ref[1])


def _pe(edge_weight, We, be):
    ew2d = edge_weight.reshape(E // D, D)
    coefs = jnp.concatenate([We.reshape(1), be.reshape(1)])
    out = pl.pallas_call(
        _pe_body,
        in_specs=[
            pl.BlockSpec(memory_space=pltpu.VMEM),
            pl.BlockSpec(memory_space=pltpu.SMEM),
        ],
        out_shape=jax.ShapeDtypeStruct((E // D, D), jnp.float32),
    )(ew2d, coefs)
    return out.reshape(E)


def _edge_body(q_hbm, kv_hbm, src_hbm, dst_hbm, pe_hbm,
               msg_hbm, den0_hbm, den1_hbm,
               sblk0, sblk1, dblk0, dblk1, pblk0, pblk1,
               qb0, qb1, kvb0, kvb1, m0, m1, es0, es1, ix0, ix1,
               acc_msg, acc_den,
               gsem0, gsem1, ssem0, ssem1, bsem0, bsem1):
    c = lax.axis_index("c")
    s = lax.axis_index("s")
    wid = c * NS + s
    tile_base = wid * (NSB * BLK)

    sblk = (sblk0, sblk1)
    dblk = (dblk0, dblk1)
    pblk = (pblk0, pblk1)
    qbuf = (qb0, qb1)
    kvbuf = (kvb0, kvb1)
    mr = (m0, m1)
    es = (es0, es1)
    ix = (ix0, ix1)
    gsem = (gsem0, gsem1)
    ssem = (ssem0, ssem1)
    bsem = (bsem0, bsem1)

    lane = jax.lax.iota(jnp.int32, L)
    zero16 = jnp.zeros((L,), jnp.float32)

    @pl.loop(0, B)
    def _(r):
        for cc in range(D // L):
            m0[r, pl.ds(cc * L, L)] = zero16
    for cc in range(B // L):
        es0[pl.ds(cc * L, L)] = zero16

    for j in range(40):
        pltpu.async_copy(m0.at[pl.ds(0, L)],
                         acc_msg.at[pl.ds(s * 624 + j * L, L)], gsem0)
    for j in range(40):
        pltpu.make_async_copy(q_hbm.at[pl.ds(0, L)], m0.at[pl.ds(0, L)],
                              gsem0).wait()
    for j in range(40):
        pltpu.async_copy(es0.at[pl.ds(0, L)],
                         acc_den.at[pl.ds(s * 624 + j * L, L)], gsem1)
    for j in range(40):
        pltpu.make_async_copy(pe_hbm.at[pl.ds(0, L)], es0.at[pl.ds(0, L)],
                              gsem1).wait()

    plsc.subcore_barrier()

    def issue_block(sb, t):
        off = tile_base + sb * BLK
        pltpu.async_copy(src_hbm.at[pl.ds(off, BLK)], sblk[t], bsem[t])
        pltpu.async_copy(dst_hbm.at[pl.ds(off, BLK)], dblk[t], bsem[t])
        pltpu.async_copy(pe_hbm.at[pl.ds(off, BLK)], pblk[t], bsem[t])

    def wait_block(t):
        pltpu.make_async_copy(src_hbm.at[pl.ds(0, BLK)], sblk[t],
                              bsem[t]).wait()
        pltpu.make_async_copy(src_hbm.at[pl.ds(0, BLK)], dblk[t],
                              bsem[t]).wait()
        pltpu.make_async_copy(pe_hbm.at[pl.ds(0, BLK)], pblk[t],
                              bsem[t]).wait()

    def issue_gathers(kb, t, p):
        pltpu.async_copy(q_hbm.at[dblk[t].at[pl.ds(kb * B, B)]], qbuf[p],
                         gsem[p])
        pltpu.async_copy(kv_hbm.at[sblk[t].at[pl.ds(kb * B, B)]], kvbuf[p],
                         gsem[p])

    def wait_gathers(p):
        pltpu.make_async_copy(q_hbm.at[pl.ds(0, B)], qbuf[p],
                              gsem[p]).wait()
        pltpu.make_async_copy(kv_hbm.at[pl.ds(0, B)], kvbuf[p],
                              gsem[p]).wait()

    def issue_scatters(p):
        pltpu.async_copy(mr[p], acc_msg.at[ix[p]], ssem[p], add=True)
        pltpu.async_copy(es[p], acc_den.at[ix[p]], ssem[p], add=True)

    def wait_scatters(p):
        pltpu.make_async_copy(q_hbm.at[pl.ds(0, B)], mr[p], ssem[p]).wait()
        pltpu.make_async_copy(pe_hbm.at[pl.ds(0, B)], es[p], ssem[p]).wait()

    def compute(kb, t, p):
        qb = qbuf[p]
        kvb = kvbuf[p]

        @pl.loop(0, B // L)
        def _(g):
            svec = jnp.zeros((L,), jnp.float32)
            for e in range(L):
                row = g * L + e
                a16 = jnp.zeros((L,), jnp.float32)
                for ch in range(D // L):
                    w = kvb[row, pl.ds(ch * L, L)]
                    kf = plsc.bitcast(w << 16, jnp.float32)
                    a16 = a16 + qb[row, pl.ds(ch * L, L)] * kf
                svec = jnp.where(lane == e, jnp.sum(a16), svec)
            esv = (jnp.exp(svec * SCALE) *
                   pblk[t][pl.ds(kb * B + g * L, L)])
            es[p][pl.ds(g * L, L)] = esv
            ix[p][pl.ds(g * L, L)] = dblk[t][pl.ds(kb * B + g * L, L)]
            for e in range(L):
                row = g * L + e
                se = jnp.broadcast_to(
                    jnp.sum(jnp.where(lane == e, esv, 0.0)), (L,))
                for ch in range(D // L):
                    w = kvb[row, pl.ds(ch * L, L)]
                    vf = plsc.bitcast(w & jnp.int32(-65536), jnp.float32)
                    mr[p][row, pl.ds(ch * L, L)] = vf * se

    issue_block(0, 0)
    issue_block(1, 1)
    wait_block(0)
    issue_gathers(0, 0, 0)
    issue_gathers(1, 0, 1)

    @pl.loop(0, NSB // 2)
    def _(sbj):
        for t in (0, 1):
            sbi = sbj * 2 + t

            @pl.loop(0, KSB // 2)
            def _(jj):
                for p in (0, 1):
                    i = jj * 2 + p
                    gi = sbi * KSB + i
                    wait_gathers(p)

                    @pl.when(gi >= 2)
                    def _():
                        wait_scatters(p)

                    compute(i, t, p)
                    issue_scatters(p)

                    @pl.when(jj < KSB // 2 - 1)
                    def _():
                        issue_gathers(i + 2, t, p)

            @pl.when(sbi + 1 < NSB)
            def _():
                wait_block(1 - t)
                issue_gathers(0, 1 - t, 0)
                issue_gathers(1, 1 - t, 1)

            @pl.when(sbi + 2 < NSB)
            def _():
                issue_block(sbi + 2, t)

    wait_scatters(0)
    wait_scatters(1)

    plsc.subcore_barrier()

    pltpu.sync_copy(acc_msg.at[pl.ds(s * 624, 624)],
                    msg_hbm.at[c, pl.ds(s * 624, 624)])

    @pl.when(s == NS - 1)
    def _():
        pltpu.sync_copy(acc_msg.at[pl.ds(9984, 16)],
                        msg_hbm.at[c, pl.ds(9984, 16)])

    @pl.when(s < 10)
    def _():
        @pl.when(c == 0)
        def _():
            pltpu.sync_copy(acc_den.at[pl.ds(s * 1024, 1024)],
                            den0_hbm.at[pl.ds(s * 1024, 1024)])

        @pl.when(c == 1)
        def _():
            pltpu.sync_copy(acc_den.at[pl.ds(s * 1024, 1024)],
                            den1_hbm.at[pl.ds(s * 1024, 1024)])


def _sc_compiler_params():
    cp = pltpu.CompilerParams()
    if "needs_layout_passes" in pltpu.CompilerParams.__dataclass_fields__:
        cp = dataclasses.replace(cp, needs_layout_passes=False)
    return cp


def _edge(q, kv, src, dst, pe):
    mesh = plsc.VectorSubcoreMesh(core_axis_name="c", subcore_axis_name="s")
    kern = pl.kernel(
        _edge_body,
        out_type=[
            jax.ShapeDtypeStruct((NC, N, D), jnp.float32),
            jax.ShapeDtypeStruct((NDEN,), jnp.float32),
            jax.ShapeDtypeStruct((NDEN,), jnp.float32),
        ],
        mesh=mesh,
        compiler_params=_sc_compiler_params(),
        scratch_types=(
            [pltpu.VMEM((BLK,), jnp.int32)] * 4 +
            [pltpu.VMEM((BLK,), jnp.float32)] * 2 +
            [pltpu.VMEM((B, D), jnp.float32)] * 2 +
            [pltpu.VMEM((B, D), jnp.int32)] * 2 +
            [pltpu.VMEM((B, D), jnp.float32)] * 2 +
            [pltpu.VMEM((B,), jnp.float32)] * 2 +
            [pltpu.VMEM((B,), jnp.int32)] * 2 +
            [pltpu.VMEM_SHARED((N, D), jnp.float32),
             pltpu.VMEM_SHARED((NDEN,), jnp.float32)] +
            [pltpu.SemaphoreType.DMA] * 6
        ),
    )
    return kern(q, kv, src, dst, pe)


def _ln(h, g_ref, b_ref):
    mu = jnp.mean(h, axis=-1, keepdims=True)
    var = jnp.mean((h - mu) ** 2, axis=-1, keepdims=True)
    return (h - mu) * jax.lax.rsqrt(var + 1e-5) * g_ref[...] + b_ref[...]


def _post_body(x_ref, p_ref, d0_ref, d1_ref, wself_ref, bself_ref,
               g1_ref, b1_ref, g2_ref, b2_ref, wf1_ref, bf1_ref,
               wf2_ref, bf2_ref, y_ref):
    xb = x_ref[...]
    mm = functools.partial(jnp.dot, preferred_element_type=jnp.float32,
                           precision=lax.Precision.HIGHEST)
    num = p_ref[0] + p_ref[1]
    den = d0_ref[...] + d1_ref[...]
    out = num / jnp.clip(den, 1e-12, None)
    h = _ln(xb + out + mm(xb, wself_ref[...].T) + bself_ref[...],
            g1_ref, b1_ref)
    f1 = mm(h, wf1_ref[...].T) + bf1_ref[...]
    gelu = 0.5 * f1 * (1.0 + lax.erf(f1 * (2.0 ** -0.5)))
    f2 = mm(gelu, wf2_ref[...].T) + bf2_ref[...]
    y_ref[...] = _ln(h + f2, g2_ref, b2_ref)


def _post(x, msg, den0, den1, Wself, bself, ln1_g, ln1_b, ln2_g, ln2_b,
          Wf1, bf1, Wf2, bf2):
    full = lambda shape: pl.BlockSpec(shape, lambda i: (0,) * len(shape))
    return pl.pallas_call(
        _post_body,
        grid=(N // ROW_BLK,),
        in_specs=[
            pl.BlockSpec((ROW_BLK, D), lambda i: (i, 0)),
            pl.BlockSpec((NC, ROW_BLK, D), lambda i: (0, i, 0)),
            pl.BlockSpec((ROW_BLK, 1), lambda i: (i, 0)),
            pl.BlockSpec((ROW_BLK, 1), lambda i: (i, 0)),
            full((D, D)), full((1, D)),
            full((1, D)), full((1, D)),
            full((1, D)), full((1, D)),
            full((2 * D, D)), full((1, 2 * D)),
            full((D, 2 * D)), full((1, D)),
        ],
        out_specs=pl.BlockSpec((ROW_BLK, D), lambda i: (i, 0)),
        out_shape=jax.ShapeDtypeStruct((N, D), jnp.float32),
    )(x, msg, den0, den1, Wself, bself.reshape(1, D),
      ln1_g.reshape(1, D), ln1_b.reshape(1, D),
      ln2_g.reshape(1, D), ln2_b.reshape(1, D),
      Wf1, bf1.reshape(1, 2 * D), Wf2, bf2.reshape(1, D))


def kernel(x, edge_index, edge_weight, Wq, bq, Wk, bk, Wv, bv, Wself, bself,
           We, be, ln1_g, ln1_b, ln2_g, ln2_b, Wf1, bf1, Wf2, bf2):
    x_pad = jnp.pad(x, ((0, NPAD - N), (0, 0)))
    q, kv = _qkv(x_pad, Wq, bq, Wk, bk, Wv, bv)
    pe = _pe(edge_weight, We, be)

    pad = E_PAD - E
    fill = (jnp.arange(pad, dtype=jnp.int32) * 37) % N
    src = jnp.concatenate([edge_index[0], fill])
    dst = jnp.concatenate([edge_index[1], fill])
    pe_pad = jnp.pad(pe, (0, pad))

    msg, den0, den1 = _edge(q, kv, src, dst, pe_pad)
    den0c = den0[:N].reshape(N, 1)
    den1c = den1[:N].reshape(N, 1)

    return _post(x, msg, den0c, den1c, Wself, bself, ln1_g, ln1_b,
                 ln2_g, ln2_b, Wf1, bf1, Wf2, bf2)
